# baseline (speedup 1.0000x reference)
import jax
import jax.numpy as jnp
from jax.experimental import pallas as pl
from jax.experimental.pallas import tpu as pltpu

D_MODEL = 2048
N_EXPERTS = 16
TOP_K = 2

CHUNK = 512
NBUF = 12


def _router_kernel(x_hbm, w_ref, gate_ref, probs_ref, buf, sem):
    i = pl.program_id(0)
    nchunk = pl.num_programs(0)

    def issue(c):
        slot = jax.lax.rem(c, NBUF)
        pltpu.make_async_copy(
            x_hbm.at[pl.ds(c * CHUNK, CHUNK), :], buf.at[slot], sem.at[slot]
        ).start()

    @pl.when(i == 0)
    def _prologue():
        for c in range(NBUF):
            issue(jnp.int32(c))

    slot = jax.lax.rem(i, NBUF)
    pltpu.make_async_copy(
        x_hbm.at[pl.ds(i * CHUNK, CHUNK), :], buf.at[slot], sem.at[slot]
    ).wait()

    x = buf[slot]
    w = w_ref[...]
    logits = jax.lax.dot_general(
        w, x, (((1,), (1,)), ((), ())), preferred_element_type=jnp.float32
    )

    m = jnp.max(logits, axis=0, keepdims=True)
    e = jnp.exp(logits - m)
    z = jnp.sum(e, axis=0, keepdims=True)
    probs = e / z
    probs_ref[...] = probs

    rows = jax.lax.broadcasted_iota(jnp.int32, probs.shape, 0)
    big = jnp.int32(N_EXPERTS)

    p1 = jnp.max(probs, axis=0, keepdims=True)
    i1 = jnp.min(jnp.where(probs >= p1, rows, big), axis=0, keepdims=True)
    masked = jnp.where(rows == i1, -jnp.inf, probs)
    p2 = jnp.max(masked, axis=0, keepdims=True)
    i2 = jnp.min(jnp.where(masked >= p2, rows, big), axis=0, keepdims=True)

    denom = p1 + p2 + 1e-09
    zero = jnp.zeros((4, CHUNK), jnp.float32)
    gate_ref[...] = jnp.concatenate(
        [i1.astype(jnp.float32), i2.astype(jnp.float32), p1 / denom, p2 / denom, zero],
        axis=0,
    )

    @pl.when(i + NBUF < nchunk)
    def _lookahead():
        issue(i + NBUF)


def kernel(x, W):
    n = x.shape[0]
    grid = (n // CHUNK,)
    out_shapes = (
        jax.ShapeDtypeStruct((8, n), jnp.float32),
        jax.ShapeDtypeStruct((N_EXPERTS, n), jnp.float32),
    )
    gate_t, probs_t = pl.pallas_call(
        _router_kernel,
        grid=grid,
        in_specs=[
            pl.BlockSpec(memory_space=pltpu.HBM),
            pl.BlockSpec((N_EXPERTS, D_MODEL), lambda i: (0, 0)),
        ],
        out_specs=(
            pl.BlockSpec((8, CHUNK), lambda i: (0, i)),
            pl.BlockSpec((N_EXPERTS, CHUNK), lambda i: (0, i)),
        ),
        out_shape=out_shapes,
        scratch_shapes=[
            pltpu.VMEM((NBUF, CHUNK, D_MODEL), jnp.float32),
            pltpu.SemaphoreType.DMA((NBUF,)),
        ],
        compiler_params=pltpu.CompilerParams(
            dimension_semantics=("arbitrary",),
        ),
    )(x, W)
    top_idx = gate_t[:TOP_K].T.astype(jnp.int32)
    weights = gate_t[TOP_K:2 * TOP_K].T
    probs = probs_t.T
    return (top_idx, weights, probs)


# speedup vs baseline: 1.0037x; 1.0037x over previous
import jax
import jax.numpy as jnp
from jax.experimental import pallas as pl
from jax.experimental.pallas import tpu as pltpu

D_MODEL = 2048
N_EXPERTS = 16
TOP_K = 2

CHUNK = 256
NBUF = 16


def _router_kernel(x_hbm, w_ref, gate_ref, probs_ref, buf, sem):
    i = pl.program_id(0)
    nchunk = pl.num_programs(0)

    def issue(c):
        slot = jax.lax.rem(c, NBUF)
        pltpu.make_async_copy(
            x_hbm.at[pl.ds(c * CHUNK, CHUNK), :], buf.at[slot], sem.at[slot]
        ).start()

    @pl.when(i == 0)
    def _prologue():
        for c in range(NBUF):
            issue(jnp.int32(c))

    slot = jax.lax.rem(i, NBUF)
    pltpu.make_async_copy(
        x_hbm.at[pl.ds(i * CHUNK, CHUNK), :], buf.at[slot], sem.at[slot]
    ).wait()

    x = buf[slot]
    w = w_ref[...]
    logits = jax.lax.dot_general(
        w, x, (((1,), (1,)), ((), ())), preferred_element_type=jnp.float32
    )

    m = jnp.max(logits, axis=0, keepdims=True)
    e = jnp.exp(logits - m)
    z = jnp.sum(e, axis=0, keepdims=True)
    probs = e / z
    probs_ref[...] = probs

    rows = jax.lax.broadcasted_iota(jnp.int32, probs.shape, 0)
    big = jnp.int32(N_EXPERTS)

    p1 = jnp.max(probs, axis=0, keepdims=True)
    i1 = jnp.min(jnp.where(probs >= p1, rows, big), axis=0, keepdims=True)
    masked = jnp.where(rows == i1, -jnp.inf, probs)
    p2 = jnp.max(masked, axis=0, keepdims=True)
    i2 = jnp.min(jnp.where(masked >= p2, rows, big), axis=0, keepdims=True)

    denom = p1 + p2 + 1e-09
    zero = jnp.zeros((4, CHUNK), jnp.float32)
    gate_ref[...] = jnp.concatenate(
        [i1.astype(jnp.float32), i2.astype(jnp.float32), p1 / denom, p2 / denom, zero],
        axis=0,
    )

    @pl.when(i + NBUF < nchunk)
    def _lookahead():
        issue(i + NBUF)


def kernel(x, W):
    n = x.shape[0]
    grid = (n // CHUNK,)
    out_shapes = (
        jax.ShapeDtypeStruct((8, n), jnp.float32),
        jax.ShapeDtypeStruct((N_EXPERTS, n), jnp.float32),
    )
    gate_t, probs_t = pl.pallas_call(
        _router_kernel,
        grid=grid,
        in_specs=[
            pl.BlockSpec(memory_space=pltpu.HBM),
            pl.BlockSpec((N_EXPERTS, D_MODEL), lambda i: (0, 0)),
        ],
        out_specs=(
            pl.BlockSpec((8, CHUNK), lambda i: (0, i)),
            pl.BlockSpec((N_EXPERTS, CHUNK), lambda i: (0, i)),
        ),
        out_shape=out_shapes,
        scratch_shapes=[
            pltpu.VMEM((NBUF, CHUNK, D_MODEL), jnp.float32),
            pltpu.SemaphoreType.DMA((NBUF,)),
        ],
        compiler_params=pltpu.CompilerParams(
            dimension_semantics=("arbitrary",),
        ),
    )(x, W)
    top_idx = gate_t[:TOP_K].T.astype(jnp.int32)
    weights = gate_t[TOP_K:2 * TOP_K].T
    probs = probs_t.T
    return (top_idx, weights, probs)
